# baseline (speedup 1.0000x reference)
import functools

import jax
import jax.numpy as jnp
from jax import lax
from jax.experimental import pallas as pl
from jax.experimental.pallas import tpu as pltpu
from jax.experimental.pallas import tpu_sc as plsc

_CHUNK = 1024


@functools.partial(jax.jit, static_argnames=("B", "D"))
def _gather_rows(idx_flat, W, B, D):
    info = plsc.get_sparse_core_info()
    NC, NS = info.num_cores, info.num_subcores
    NW = NC * NS
    b_per_w = B // NW
    n_chunks = b_per_w // _CHUNK
    mesh = plsc.VectorSubcoreMesh(core_axis_name="c", subcore_axis_name="s")

    @functools.partial(
        pl.kernel,
        mesh=mesh,
        compiler_params=pltpu.CompilerParams(use_tc_tiling_on_sc=False),
        out_type=jax.ShapeDtypeStruct((B, D), jnp.float32),
        scratch_types=[
            pltpu.VMEM((_CHUNK,), jnp.int32),
            pltpu.VMEM((_CHUNK, D), jnp.float32),
            pltpu.SemaphoreType.DMA,
        ],
    )
    def k(table_hbm, idx_hbm, out_hbm, idx_v, rows_v, sem):
        wid = lax.axis_index("s") * NC + lax.axis_index("c")
        base = wid * b_per_w

        def body(i, carry):
            off = base + i * _CHUNK
            pltpu.sync_copy(idx_hbm.at[pl.ds(off, _CHUNK)], idx_v)
            pltpu.async_copy(table_hbm.at[idx_v], rows_v, sem).wait()
            pltpu.sync_copy(rows_v, out_hbm.at[pl.ds(off, _CHUNK)])
            return carry

        lax.fori_loop(0, n_chunks, body, 0)

    return k(W, idx_flat)


def kernel(inp, W):
    B = inp.shape[0] * inp.shape[1]
    D = W.shape[1]
    idx_flat = inp.reshape(B).astype(jnp.int32)
    out = _gather_rows(idx_flat, W, B, D)
    return out.reshape(inp.shape[0], inp.shape[1], D)


# speedup vs baseline: 1.0092x; 1.0092x over previous
import functools

import jax
import jax.numpy as jnp
from jax import lax
from jax.experimental import pallas as pl
from jax.experimental.pallas import tpu as pltpu
from jax.experimental.pallas import tpu_sc as plsc

_CHUNK = 800


@functools.partial(jax.jit, static_argnames=("B", "D"))
def _gather_rows(idx_flat, W, B, D):
    info = plsc.get_sparse_core_info()
    NC, NS = info.num_cores, info.num_subcores
    NW = NC * NS
    b_per_w = B // NW
    n_chunks = b_per_w // _CHUNK
    C = _CHUNK
    mesh = plsc.VectorSubcoreMesh(core_axis_name="c", subcore_axis_name="s")

    @functools.partial(
        pl.kernel,
        mesh=mesh,
        compiler_params=pltpu.CompilerParams(use_tc_tiling_on_sc=False),
        out_type=jax.ShapeDtypeStruct((B, D), jnp.float32),
        scratch_types=[
            pltpu.VMEM((b_per_w,), jnp.int32),
            pltpu.VMEM((C, D), jnp.float32),
            pltpu.VMEM((C, D), jnp.float32),
            pltpu.SemaphoreType.DMA,
            pltpu.SemaphoreType.DMA,
            pltpu.SemaphoreType.DMA,
            pltpu.SemaphoreType.DMA,
        ],
    )
    def k(table_hbm, idx_hbm, out_hbm, idx_v, rows0, rows1, gs0, gs1, ws0, ws1):
        wid = lax.axis_index("s") * NC + lax.axis_index("c")
        base = wid * b_per_w
        pltpu.sync_copy(idx_hbm.at[pl.ds(base, b_per_w)], idx_v)

        def g_start(g, rows, sem):
            pltpu.async_copy(table_hbm.at[idx_v.at[pl.ds(g * C, C)]], rows, sem)

        def g_wait(rows, sem):
            pltpu.make_async_copy(
                table_hbm.at[idx_v.at[pl.ds(0, C)]], rows, sem
            ).wait()

        def w_start(g, rows, sem):
            pltpu.async_copy(rows, out_hbm.at[pl.ds(base + g * C, C)], sem)

        def w_wait(rows, sem):
            pltpu.make_async_copy(rows, out_hbm.at[pl.ds(base, C)], sem).wait()

        g_start(0, rows0, gs0)
        g_start(1, rows1, gs1)

        def body(j, carry):
            g0 = 2 * j
            g1 = g0 + 1
            g_wait(rows0, gs0)
            w_start(g0, rows0, ws0)
            g_wait(rows1, gs1)
            w_start(g1, rows1, ws1)
            w_wait(rows0, ws0)
            g_start(g0 + 2, rows0, gs0)
            w_wait(rows1, ws1)
            g_start(g1 + 2, rows1, gs1)
            return carry

        lax.fori_loop(0, n_chunks // 2 - 1, body, 0)

        g_wait(rows0, gs0)
        w_start(n_chunks - 2, rows0, ws0)
        g_wait(rows1, gs1)
        w_start(n_chunks - 1, rows1, ws1)
        w_wait(rows0, ws0)
        w_wait(rows1, ws1)

    return k(W, idx_flat)


def kernel(inp, W):
    B = inp.shape[0] * inp.shape[1]
    D = W.shape[1]
    idx_flat = inp.reshape(B).astype(jnp.int32)
    out = _gather_rows(idx_flat, W, B, D)
    return out.reshape(inp.shape[0], inp.shape[1], D)
